# baseline (speedup 1.0000x reference)
import functools

import jax
import jax.numpy as jnp
from jax import lax
from jax.experimental import pallas as pl
from jax.experimental.pallas import tpu as pltpu
from jax.experimental.pallas import tpu_sc as plsc

N_ENVS = 100
N_CLASSES = 1000
Z = 64
D = 2 * Z
BATCH = 4096

_info = plsc.get_sparse_core_info()
_NC, _NS, _L = _info.num_cores, _info.num_subcores, _info.num_lanes
_NW = _NC * _NS
_BPW = BATCH // _NW
_NBUF = 4


def _sc_gather_body(y_hbm, e_hbm, mu_c_hbm, cov_c_hbm, mu_s_hbm, cov_s_hbm,
                    muc_out, mus_out, covc_out, covs_out,
                    y_v, e_v, flat_v, muc_v, covc_v, mus_v, covs_v, sem):
    wid = lax.axis_index("s") * _NC + lax.axis_index("c")
    base = wid * _BPW
    pltpu.sync_copy(y_hbm.at[pl.ds(base, _BPW)], y_v)
    pltpu.sync_copy(e_hbm.at[pl.ds(base, _BPW)], e_v)
    for j in range(_BPW // _L):
        sl = pl.ds(j * _L, _L)
        flat_v[sl] = y_v[sl] * N_ENVS + e_v[sl]
    c1 = pltpu.make_async_copy(mu_c_hbm.at[e_v], muc_v, sem)
    c2 = pltpu.make_async_copy(cov_c_hbm.at[e_v], covc_v, sem)
    c3 = pltpu.make_async_copy(mu_s_hbm.at[flat_v], mus_v, sem)
    c4 = pltpu.make_async_copy(cov_s_hbm.at[flat_v], covs_v, sem)
    c1.start(); c2.start(); c3.start(); c4.start()
    c1.wait(); c2.wait(); c3.wait(); c4.wait()
    rows = pl.ds(base, _BPW)
    pltpu.sync_copy(muc_v, muc_out.at[rows])
    pltpu.sync_copy(mus_v, mus_out.at[rows])
    pltpu.sync_copy(covc_v, covc_out.at[rows])
    pltpu.sync_copy(covs_v, covs_out.at[rows])


_sc_gather = functools.partial(
    pl.kernel,
    mesh=plsc.VectorSubcoreMesh(core_axis_name="c", subcore_axis_name="s"),
    out_type=[jax.ShapeDtypeStruct((BATCH, Z), jnp.float32)] * 4,
    scratch_types=[
        pltpu.VMEM((_BPW,), jnp.int32),
        pltpu.VMEM((_BPW,), jnp.int32),
        pltpu.VMEM((_BPW,), jnp.int32),
        pltpu.VMEM((_BPW, Z), jnp.float32),
        pltpu.VMEM((_BPW, Z), jnp.float32),
        pltpu.VMEM((_BPW, Z), jnp.float32),
        pltpu.VMEM((_BPW, Z), jnp.float32),
        pltpu.SemaphoreType.DMA,
    ],
    compiler_params=pltpu.CompilerParams(use_tc_tiling_on_sc=False),
)(_sc_gather_body)


def _sc_write_body(cov_hbm, out_hbm, cov_v, b0, b1, b2, b3, sems):
    bufs = (b0, b1, b2, b3)
    wid = lax.axis_index("s") * _NC + lax.axis_index("c")
    base = wid * _BPW
    pltpu.sync_copy(cov_hbm.at[pl.ds(base, _BPW)], cov_v)
    zeros = jnp.zeros((_L,), jnp.float32)

    def _zero(i, _):
        for b in range(_NBUF):
            for j in range(D // _L):
                bufs[b][i, pl.ds(j * _L, _L)] = zeros
        return 0

    lax.fori_loop(0, D, _zero, 0)

    def _rows(g, _):
        for b in range(_NBUF):
            r = g * _NBUF + b

            @pl.when(g > 0)
            def _():
                pltpu.make_async_copy(
                    bufs[b], out_hbm.at[base + r - _NBUF], sems.at[b]).wait()

            for j in range(D // _L):
                ids = lax.iota(jnp.int32, _L) + j * _L
                vals = cov_v[r, pl.ds(j * _L, _L)]
                plsc.store_scatter(bufs[b], [ids, ids], vals)
            pltpu.make_async_copy(
                bufs[b], out_hbm.at[base + r], sems.at[b]).start()
        return 0

    lax.fori_loop(0, _BPW // _NBUF, _rows, 0)
    for b in range(_NBUF):
        pltpu.make_async_copy(
            bufs[b], out_hbm.at[base + _BPW - _NBUF + b], sems.at[b]).wait()


_sc_write = functools.partial(
    pl.kernel,
    mesh=plsc.VectorSubcoreMesh(core_axis_name="c", subcore_axis_name="s"),
    out_type=jax.ShapeDtypeStruct((BATCH, D, D), jnp.float32),
    scratch_types=(
        [pltpu.VMEM((_BPW, D), jnp.float32)]
        + [pltpu.VMEM((D, D), jnp.float32)] * _NBUF
        + [pltpu.SemaphoreType.DMA((_NBUF,))]
    ),
    compiler_params=pltpu.CompilerParams(needs_layout_passes=False),
)(_sc_write_body)


def _tc_small_body(muc_ref, mus_ref, covc_ref, covs_ref, mu_ref, cov_ref):
    mu_ref[...] = jnp.concatenate([muc_ref[...], mus_ref[...]], axis=-1)
    cov_ref[...] = jax.nn.softplus(
        jnp.concatenate([covc_ref[...], covs_ref[...]], axis=-1))


def _tc_small(muc, mus, covc, covs):
    bb = 1024
    half = pl.BlockSpec((bb, Z), lambda b: (b, 0))
    full = pl.BlockSpec((bb, D), lambda b: (b, 0))
    return pl.pallas_call(
        _tc_small_body,
        grid=(BATCH // bb,),
        in_specs=[half, half, half, half],
        out_specs=[full, full],
        out_shape=[jax.ShapeDtypeStruct((BATCH, D), jnp.float32)] * 2,
    )(muc, mus, covc, covs)


def kernel(y, e, mu_causal, cov_causal, mu_spurious, cov_spurious):
    y_flat = y[:, 0].astype(jnp.int32)
    e_flat = e[:, 0].astype(jnp.int32)
    mu_s2d = mu_spurious.reshape(N_CLASSES * N_ENVS, Z)
    cov_s2d = cov_spurious.reshape(N_CLASSES * N_ENVS, Z)
    muc, mus, covc, covs = _sc_gather(y_flat, e_flat, mu_causal, cov_causal,
                                      mu_s2d, cov_s2d)
    mu, cov_sp = _tc_small(muc, mus, covc, covs)
    cov_mat = _sc_write(cov_sp)
    return mu, cov_mat


# speedup vs baseline: 1.1106x; 1.1106x over previous
import functools

import jax
import jax.numpy as jnp
from jax import lax
from jax.experimental import pallas as pl
from jax.experimental.pallas import tpu as pltpu
from jax.experimental.pallas import tpu_sc as plsc

N_ENVS = 100
N_CLASSES = 1000
Z = 64
BATCH = 4096

_info = plsc.get_sparse_core_info()
_NC, _NS, _L = _info.num_cores, _info.num_subcores, _info.num_lanes
_NW = _NC * _NS
_BPW = BATCH // _NW
_NCH = _BPW // _L


def _sc_gather_body(y_hbm, e_hbm, mu_c_hbm, cov_c_hbm, mu_s_hbm, cov_s_hbm,
                    muc_out, mus_out, covc_out, covs_out,
                    y_v, e_v, muc_tab, covc_tab, msl, csl,
                    muc_v, mus_v, covc_v, covs_v, sem):
    wid = lax.axis_index("s") * _NC + lax.axis_index("c")
    base = wid * _BPW
    pltpu.sync_copy(y_hbm.at[pl.ds(base, _BPW)], y_v)
    pltpu.sync_copy(e_hbm.at[pl.ds(base, _BPW)], e_v)
    pltpu.sync_copy(mu_c_hbm, muc_tab)
    pltpu.sync_copy(cov_c_hbm, covc_tab)
    lanes = lax.iota(jnp.int32, _L)
    zeros = jnp.zeros((_L,), jnp.int32)

    def _chunk(k, _):
        y16 = y_v[pl.ds(k * _L, _L)]
        e16 = e_v[pl.ds(k * _L, _L)]
        copies = []
        for lane in range(_L):
            y_s = jnp.sum(jnp.where(lanes == lane, y16, zeros))
            e_s = jnp.sum(jnp.where(lanes == lane, e16, zeros))
            m8 = pl.multiple_of((e_s // 8) * 8, 8)
            c1 = pltpu.make_async_copy(
                mu_s_hbm.at[y_s, pl.ds(m8, 8), :], msl.at[lane], sem)
            c2 = pltpu.make_async_copy(
                cov_s_hbm.at[y_s, pl.ds(m8, 8), :], csl.at[lane], sem)
            c1.start(); c2.start()
            copies.append(c1); copies.append(c2)
        for c in copies:
            c.wait()
        for lane in range(_L):
            e_s = jnp.sum(jnp.where(lanes == lane, e16, zeros))
            off = lax.rem(e_s, 8)
            r = k * _L + lane
            for j in range(Z // _L):
                sl = pl.ds(j * _L, _L)
                mus_v[r, sl] = msl[lane, off, sl]
                covs_v[r, sl] = csl[lane, off, sl]
                muc_v[r, sl] = muc_tab[e_s, sl]
                covc_v[r, sl] = covc_tab[e_s, sl]
        return 0

    lax.fori_loop(0, _NCH, _chunk, 0)
    rows = pl.ds(base, _BPW)
    pltpu.sync_copy(muc_v, muc_out.at[rows])
    pltpu.sync_copy(mus_v, mus_out.at[rows])
    pltpu.sync_copy(covc_v, covc_out.at[rows])
    pltpu.sync_copy(covs_v, covs_out.at[rows])


_sc_gather = functools.partial(
    pl.kernel,
    mesh=plsc.VectorSubcoreMesh(core_axis_name="c", subcore_axis_name="s"),
    out_type=[jax.ShapeDtypeStruct((BATCH, Z), jnp.float32)] * 4,
    scratch_types=[
        pltpu.VMEM((_BPW,), jnp.int32),
        pltpu.VMEM((_BPW,), jnp.int32),
        pltpu.VMEM((N_ENVS, Z), jnp.float32),
        pltpu.VMEM((N_ENVS, Z), jnp.float32),
        pltpu.VMEM((_L, 8, Z), jnp.float32),
        pltpu.VMEM((_L, 8, Z), jnp.float32),
        pltpu.VMEM((_BPW, Z), jnp.float32),
        pltpu.VMEM((_BPW, Z), jnp.float32),
        pltpu.VMEM((_BPW, Z), jnp.float32),
        pltpu.VMEM((_BPW, Z), jnp.float32),
        pltpu.SemaphoreType.DMA,
    ],
    compiler_params=pltpu.CompilerParams(use_tc_tiling_on_sc=True,
                                         needs_layout_passes=False),
)(_sc_gather_body)


_BB = 256


def _tc_body(muc_ref, mus_ref, covc_ref, covs_ref, mu_ref, out_ref):
    mu_ref[...] = jnp.concatenate([muc_ref[...], mus_ref[...]], axis=-1)
    cov = jax.nn.softplus(
        jnp.concatenate([covc_ref[...], covs_ref[...]], axis=-1))
    eye = (lax.broadcasted_iota(jnp.int32, (2 * Z, 2 * Z), 0)
           == lax.broadcasted_iota(jnp.int32, (2 * Z, 2 * Z), 1))
    out_ref[...] = jnp.where(eye[None], cov[:, :, None], jnp.float32(0.0))


def _tc_diag(muc, mus, covc, covs):
    half = pl.BlockSpec((_BB, Z), lambda b: (b, 0))
    return pl.pallas_call(
        _tc_body,
        grid=(BATCH // _BB,),
        in_specs=[half, half, half, half],
        out_specs=[
            pl.BlockSpec((_BB, 2 * Z), lambda b: (b, 0)),
            pl.BlockSpec((_BB, 2 * Z, 2 * Z), lambda b: (b, 0, 0)),
        ],
        out_shape=[
            jax.ShapeDtypeStruct((BATCH, 2 * Z), jnp.float32),
            jax.ShapeDtypeStruct((BATCH, 2 * Z, 2 * Z), jnp.float32),
        ],
    )(muc, mus, covc, covs)


def kernel(y, e, mu_causal, cov_causal, mu_spurious, cov_spurious):
    y_flat = y[:, 0].astype(jnp.int32)
    e_flat = e[:, 0].astype(jnp.int32)
    muc, mus, covc, covs = _sc_gather(y_flat, e_flat, mu_causal, cov_causal,
                                      mu_spurious, cov_spurious)
    mu, cov_mat = _tc_diag(muc, mus, covc, covs)
    return mu, cov_mat
